# baseline (speedup 1.0000x reference)
import functools

import jax
import jax.numpy as jnp
from jax import lax
from jax.experimental import pallas as pl
from jax.experimental.pallas import tpu as pltpu
from jax.experimental.pallas import tpu_sc as plsc

_NW = 32
_NS = 16
_CHUNK = 128


def _make_deg_kernel(n_pad, c_chunks, batch):
    mesh = plsc.VectorSubcoreMesh(core_axis_name="c", subcore_axis_name="s")
    zrows = n_pad // _NS
    groups = c_chunks // batch
    brows = batch * _CHUNK

    @functools.partial(
        pl.kernel,
        mesh=mesh,
        out_type=jax.ShapeDtypeStruct((2 * n_pad, 16), jnp.float32),
        compiler_params=pltpu.CompilerParams(use_tc_tiling_on_sc=False),
        scratch_types=[
            pltpu.VMEM((groups, batch, _CHUNK), jnp.int32),
            pltpu.VMEM((_CHUNK, 16), jnp.float32),
            pltpu.VMEM((_CHUNK, 16), jnp.float32),
            pltpu.VMEM_SHARED((n_pad, 16), jnp.float32),
        ],
    )
    def k(rows_hbm, out_hbm, rows_v, ebuf, zbuf, acc):
        cid = lax.axis_index("c")
        sid = lax.axis_index("s")
        wid = cid * _NS + sid
        e0v = jnp.where(lax.iota(jnp.int32, 16) == 0, 1.0, 0.0)
        z16 = jnp.zeros((16,), jnp.float32)

        def fill(i, carry):
            ebuf[i, pl.ds(0, 16)] = e0v
            return carry

        lax.fori_loop(0, _CHUNK, fill, 0)

        def fillz(i, carry):
            zbuf[i, pl.ds(0, 16)] = z16
            return carry

        lax.fori_loop(0, _CHUNK, fillz, 0)

        def zstripe(k2, carry):
            pltpu.sync_copy(
                zbuf, acc.at[pl.ds(sid * zrows + k2 * _CHUNK, _CHUNK)])
            return carry

        lax.fori_loop(0, zrows // _CHUNK, zstripe, 0)
        pltpu.sync_copy(rows_hbm.at[wid], rows_v)
        plsc.subcore_barrier()

        def body(j, carry):
            for b in range(batch):
                pltpu.sync_copy(ebuf, acc.at[rows_v.at[j, b]], add=True)
            return carry

        lax.fori_loop(0, groups, body, 0)
        plsc.subcore_barrier()
        pltpu.sync_copy(acc.at[pl.ds(sid * zrows, zrows)],
                        out_hbm.at[pl.ds(cid * n_pad + sid * zrows, zrows)])

    return k


def _make_spmm_kernel(n_pad, c_chunks, feat, batch):
    mesh = plsc.VectorSubcoreMesh(core_axis_name="c", subcore_axis_name="s")
    zrows = n_pad // _NS
    groups = c_chunks // batch
    brows = batch * _CHUNK

    @functools.partial(
        pl.kernel,
        mesh=mesh,
        out_type=jax.ShapeDtypeStruct((2 * n_pad, feat), jnp.float32),
        compiler_params=pltpu.CompilerParams(use_tc_tiling_on_sc=False),
        scratch_types=[
            pltpu.VMEM((groups + 1, batch, _CHUNK), jnp.int32),
            pltpu.VMEM((groups, batch, _CHUNK), jnp.int32),
            pltpu.VMEM((_CHUNK, feat), jnp.float32),
            pltpu.VMEM((_CHUNK, feat), jnp.float32),
            pltpu.VMEM_SHARED((n_pad, feat), jnp.float32),
            pltpu.VMEM_SHARED((n_pad, feat), jnp.float32),
            pltpu.SemaphoreType.DMA,
            pltpu.SemaphoreType.DMA,
        ],
    )
    def k(g_hbm, rows_hbm, cols_hbm, out_hbm,
          rows_v, cols_v, gbuf0, gbuf1, acc, gtab, sem0, sem1):
        cid = lax.axis_index("c")
        sid = lax.axis_index("s")
        wid = cid * _NS + sid
        z16 = jnp.zeros((16,), jnp.float32)

        def fill(i, carry):
            for kk in range(feat // 16):
                gbuf0[i, pl.ds(16 * kk, 16)] = z16
            return carry

        lax.fori_loop(0, _CHUNK, fill, 0)

        def zstripe(k2, carry):
            pltpu.sync_copy(
                gbuf0.at[pl.ds(0, _CHUNK)],
                acc.at[pl.ds(sid * zrows + k2 * _CHUNK, _CHUNK)])
            return carry

        lax.fori_loop(0, zrows // _CHUNK, zstripe, 0)
        pltpu.sync_copy(rows_hbm.at[wid], rows_v.at[pl.ds(0, groups)])
        pltpu.sync_copy(cols_hbm.at[wid], cols_v)
        lane = lax.iota(jnp.int32, 16)
        for b in range(batch):
            for kk in range(_CHUNK // 16):
                rows_v[groups, b, pl.ds(16 * kk, 16)] = (
                    (n_pad - _CHUNK) + 16 * kk + lane)
        pltpu.sync_copy(g_hbm.at[pl.ds(sid * zrows, zrows)],
                        gtab.at[pl.ds(sid * zrows, zrows)])
        plsc.subcore_barrier()

        bufs = (gbuf0, gbuf1)
        sems = (sem0, sem1)
        pltpu.async_copy(gtab.at[rows_v.at[0, 0]], gbuf0, sem0)

        def body(j, carry):
            for b in range(batch):
                cur = b % 2
                nxt = 1 - cur
                pltpu.make_async_copy(gtab.at[rows_v.at[j, b]], bufs[cur],
                                      sems[cur]).wait()
                if b + 1 < batch:
                    pltpu.async_copy(gtab.at[rows_v.at[j, b + 1]],
                                     bufs[nxt], sems[nxt])
                else:
                    pltpu.async_copy(gtab.at[rows_v.at[j + 1, 0]],
                                     bufs[nxt], sems[nxt])
                pltpu.sync_copy(bufs[cur], acc.at[cols_v.at[j, b]],
                                add=True)
            return carry

        lax.fori_loop(0, groups, body, 0)
        pltpu.make_async_copy(gtab.at[rows_v.at[groups, 0]],
                              bufs[batch % 2], sems[batch % 2]).wait()
        plsc.subcore_barrier()
        pltpu.sync_copy(acc.at[pl.ds(sid * zrows, zrows)],
                        out_hbm.at[pl.ds(cid * n_pad + sid * zrows, zrows)])

    return k


def _d_col(degp, n, n_pad):
    deg = degp[:n, 0:1] + degp[n_pad:n_pad + n, 0:1]
    return jnp.where(deg > 0.0, lax.rsqrt(deg), 0.0)


def _psum(sp, n, n_pad):
    return sp[:n] + sp[n_pad:n_pad + n]


def _tc1a_body(x_ref, w10_ref, w11_ref, z1_ref, y1_ref):
    x = x_ref[...]
    z1_ref[...] = jnp.dot(x, w10_ref[...], preferred_element_type=jnp.float32)
    y1_ref[...] = jnp.dot(x, w11_ref[...], preferred_element_type=jnp.float32)


def _tc1b_body(n, n_pad, y1_ref, degp_ref, g1_ref):
    d = _d_col(degp_ref[...], n, n_pad)
    g1_ref[0:n, :] = d * y1_ref[...]
    g1_ref[n:n_pad, :] = jnp.zeros((n_pad - n, y1_ref.shape[1]), jnp.float32)


def _tc2_body(n, n_pad, z1_ref, sp_ref, degp_ref, b1_ref, w20_ref, w21_ref,
              z2_ref, g2_ref):
    d = _d_col(degp_ref[...], n, n_pad)
    s1 = d * _psum(sp_ref[...], n, n_pad)
    h = jnp.maximum(z1_ref[...] - s1 + b1_ref[...], 0.0)
    z2_ref[...] = jnp.dot(h, w20_ref[...], preferred_element_type=jnp.float32)
    g2_ref[0:n, :] = d * jnp.dot(h, w21_ref[...],
                                 preferred_element_type=jnp.float32)
    g2_ref[n:n_pad, :] = jnp.zeros((n_pad - n, w21_ref.shape[1]), jnp.float32)


def _tc3_body(n, n_pad, z2_ref, sp_ref, degp_ref, b2_ref, out_ref):
    d = _d_col(degp_ref[...], n, n_pad)
    o = z2_ref[...] - d * _psum(sp_ref[...], n, n_pad) + b2_ref[...]
    m = jnp.max(o, axis=1, keepdims=True)
    e = jnp.exp(o - m)
    out_ref[...] = (o - m) - jnp.log(jnp.sum(e, axis=1, keepdims=True))


def kernel(x, edge_index, W1, b1, W2, b2):
    n, in_c = x.shape
    e = edge_index.shape[1]
    hid = W1.shape[2]
    out_c = W2.shape[2]

    batch = 8
    c_chunks = batch * (-(-e // (_NW * _CHUNK * batch)))
    e_pad = _NW * c_chunks * _CHUNK
    n_pad = ((n + _CHUNK + 127) // 128) * 128

    rows = edge_index[0].astype(jnp.int32)
    cols = edge_index[1].astype(jnp.int32)
    pad = n + (jnp.arange(e_pad - e, dtype=jnp.int32) % _CHUNK)
    shape4 = (_NW, c_chunks // batch, batch, _CHUNK)
    rows3 = jnp.concatenate([rows, pad]).reshape(shape4)
    cols3 = jnp.concatenate([cols, pad]).reshape(shape4)

    degp = _make_deg_kernel(n_pad, c_chunks, batch)(rows3)

    b1r = b1.reshape(1, hid)
    b2r = b2.reshape(1, out_c)

    z1, y1 = pl.pallas_call(
        _tc1a_body,
        out_shape=(jax.ShapeDtypeStruct((n, hid), jnp.float32),
                   jax.ShapeDtypeStruct((n, hid), jnp.float32)),
    )(x, W1[0], W1[1])

    g1p = pl.pallas_call(
        functools.partial(_tc1b_body, n, n_pad),
        out_shape=jax.ShapeDtypeStruct((n_pad, hid), jnp.float32),
    )(y1, degp)

    s1p = _make_spmm_kernel(n_pad, c_chunks, hid, batch)(g1p, rows3, cols3)

    z2, g2p = pl.pallas_call(
        functools.partial(_tc2_body, n, n_pad),
        out_shape=(jax.ShapeDtypeStruct((n, out_c), jnp.float32),
                   jax.ShapeDtypeStruct((n_pad, out_c), jnp.float32)),
    )(z1, s1p, degp, b1r, W2[0], W2[1])

    s2p = _make_spmm_kernel(n_pad, c_chunks, out_c, batch)(g2p, rows3,
                                                           cols3)

    return pl.pallas_call(
        functools.partial(_tc3_body, n, n_pad),
        out_shape=jax.ShapeDtypeStruct((n, out_c), jnp.float32),
    )(z2, s2p, degp, b2r)


# speedup vs baseline: 1.0019x; 1.0019x over previous
import functools

import jax
import jax.numpy as jnp
import numpy as np
from jax import lax
from jax.experimental import pallas as pl
from jax.experimental.pallas import tpu as pltpu
from jax.experimental.pallas import tpu_sc as plsc

_NW = 32
_NS = 16
_CHUNK = 128


def _make_deg_kernel(n_pad, c_chunks, batch):
    mesh = plsc.VectorSubcoreMesh(core_axis_name="c", subcore_axis_name="s")
    zrows = n_pad // _NS
    groups = c_chunks // batch
    brows = batch * _CHUNK

    @functools.partial(
        pl.kernel,
        mesh=mesh,
        out_type=jax.ShapeDtypeStruct((2 * n_pad, 16), jnp.float32),
        compiler_params=pltpu.CompilerParams(use_tc_tiling_on_sc=False),
        scratch_types=[
            pltpu.VMEM((groups, batch, _CHUNK), jnp.int32),
            pltpu.VMEM((_CHUNK, 16), jnp.float32),
            pltpu.VMEM((_CHUNK, 16), jnp.float32),
            pltpu.VMEM_SHARED((n_pad, 16), jnp.float32),
        ],
    )
    def k(rows_hbm, out_hbm, rows_v, ebuf, zbuf, acc):
        cid = lax.axis_index("c")
        sid = lax.axis_index("s")
        wid = cid * _NS + sid
        e0v = jnp.where(lax.iota(jnp.int32, 16) == 0, 1.0, 0.0)
        z16 = jnp.zeros((16,), jnp.float32)

        def fill(i, carry):
            ebuf[i, pl.ds(0, 16)] = e0v
            return carry

        lax.fori_loop(0, _CHUNK, fill, 0)

        def fillz(i, carry):
            zbuf[i, pl.ds(0, 16)] = z16
            return carry

        lax.fori_loop(0, _CHUNK, fillz, 0)

        def zstripe(k2, carry):
            pltpu.sync_copy(
                zbuf, acc.at[pl.ds(sid * zrows + k2 * _CHUNK, _CHUNK)])
            return carry

        lax.fori_loop(0, zrows // _CHUNK, zstripe, 0)
        pltpu.sync_copy(rows_hbm.at[wid], rows_v)
        plsc.subcore_barrier()

        def body(j, carry):
            for b in range(batch):
                pltpu.sync_copy(ebuf, acc.at[rows_v.at[j, b]], add=True)
            return carry

        lax.fori_loop(0, groups, body, 0)
        plsc.subcore_barrier()
        pltpu.sync_copy(acc.at[pl.ds(sid * zrows, zrows)],
                        out_hbm.at[pl.ds(cid * n_pad + sid * zrows, zrows)])

    return k


def _make_spmm_kernel(n_pad, c_chunks, feat, batch):
    mesh = plsc.VectorSubcoreMesh(core_axis_name="c", subcore_axis_name="s")
    zrows = n_pad // _NS
    groups = c_chunks // batch
    brows = batch * _CHUNK

    @functools.partial(
        pl.kernel,
        mesh=mesh,
        out_type=jax.ShapeDtypeStruct((2 * n_pad, feat), jnp.float32),
        compiler_params=pltpu.CompilerParams(use_tc_tiling_on_sc=False),
        scratch_types=[
            pltpu.VMEM((groups + 1, batch, _CHUNK), jnp.int32),
            pltpu.VMEM((groups, batch, _CHUNK), jnp.int32),
            pltpu.VMEM((_CHUNK, feat), jnp.float32),
            pltpu.VMEM((_CHUNK, feat), jnp.float32),
            pltpu.VMEM_SHARED((n_pad, feat), jnp.float32),
            pltpu.VMEM_SHARED((n_pad, feat), jnp.float32),
            pltpu.SemaphoreType.DMA,
            pltpu.SemaphoreType.DMA,
        ],
    )
    def k(g_hbm, rows_hbm, cols_hbm, out_hbm,
          rows_v, cols_v, gbuf0, gbuf1, acc, gtab, sem0, sem1):
        cid = lax.axis_index("c")
        sid = lax.axis_index("s")
        wid = cid * _NS + sid
        z16 = jnp.zeros((16,), jnp.float32)

        def fill(i, carry):
            for kk in range(feat // 16):
                gbuf0[i, pl.ds(16 * kk, 16)] = z16
            return carry

        lax.fori_loop(0, _CHUNK, fill, 0)

        def zstripe(k2, carry):
            pltpu.sync_copy(
                gbuf0.at[pl.ds(0, _CHUNK)],
                acc.at[pl.ds(sid * zrows + k2 * _CHUNK, _CHUNK)])
            return carry

        lax.fori_loop(0, zrows // _CHUNK, zstripe, 0)
        pltpu.sync_copy(rows_hbm.at[wid], rows_v.at[pl.ds(0, groups)])
        pltpu.sync_copy(cols_hbm.at[wid], cols_v)
        lane = lax.iota(jnp.int32, 16)
        for b in range(batch):
            for kk in range(_CHUNK // 16):
                rows_v[groups, b, pl.ds(16 * kk, 16)] = (
                    (n_pad - _CHUNK) + 16 * kk + lane)
        pltpu.sync_copy(g_hbm.at[pl.ds(sid * zrows, zrows)],
                        gtab.at[pl.ds(sid * zrows, zrows)])
        plsc.subcore_barrier()

        bufs = (gbuf0, gbuf1)
        sems = (sem0, sem1)
        pltpu.async_copy(gtab.at[rows_v.at[0, 0]], gbuf0, sem0)

        def body(j, carry):
            for b in range(batch):
                cur = b % 2
                nxt = 1 - cur
                pltpu.make_async_copy(gtab.at[rows_v.at[j, b]], bufs[cur],
                                      sems[cur]).wait()
                if b + 1 < batch:
                    pltpu.async_copy(gtab.at[rows_v.at[j, b + 1]],
                                     bufs[nxt], sems[nxt])
                else:
                    pltpu.async_copy(gtab.at[rows_v.at[j + 1, 0]],
                                     bufs[nxt], sems[nxt])
                pltpu.sync_copy(bufs[cur], acc.at[cols_v.at[j, b]],
                                add=True)
            return carry

        lax.fori_loop(0, groups, body, 0)
        pltpu.make_async_copy(gtab.at[rows_v.at[groups, 0]],
                              bufs[batch % 2], sems[batch % 2]).wait()
        plsc.subcore_barrier()
        pltpu.sync_copy(acc.at[pl.ds(sid * zrows, zrows)],
                        out_hbm.at[pl.ds(cid * n_pad + sid * zrows, zrows)])

    return k


def _d_col(degp, n, n_pad):
    deg = degp[:n, 0:1] + degp[n_pad:n_pad + n, 0:1]
    return jnp.where(deg > 0.0, lax.rsqrt(deg), 0.0)


def _psum(sp, n, n_pad):
    return sp[:n] + sp[n_pad:n_pad + n]


def _tc1a_body(x_ref, w10_ref, w11_ref, z1_ref, y1_ref):
    x = x_ref[...]
    z1_ref[...] = jnp.dot(x, w10_ref[...], preferred_element_type=jnp.float32)
    y1_ref[...] = jnp.dot(x, w11_ref[...], preferred_element_type=jnp.float32)


def _tc1b_body(n, n_pad, y1_ref, degp_ref, g1_ref):
    d = _d_col(degp_ref[...], n, n_pad)
    g1_ref[0:n, :] = d * y1_ref[...]
    g1_ref[n:n_pad, :] = jnp.zeros((n_pad - n, y1_ref.shape[1]), jnp.float32)


def _tc2_body(n, n_pad, z1_ref, sp_ref, degp_ref, b1_ref, w20_ref, w21_ref,
              z2_ref, g2_ref):
    d = _d_col(degp_ref[...], n, n_pad)
    s1 = d * _psum(sp_ref[...], n, n_pad)
    h = jnp.maximum(z1_ref[...] - s1 + b1_ref[...], 0.0)
    z2_ref[...] = jnp.dot(h, w20_ref[...], preferred_element_type=jnp.float32)
    g2_ref[0:n, :] = d * jnp.dot(h, w21_ref[...],
                                 preferred_element_type=jnp.float32)
    g2_ref[n:n_pad, :] = jnp.zeros((n_pad - n, w21_ref.shape[1]), jnp.float32)


def _tc3_body(n, n_pad, z2_ref, sp_ref, degp_ref, b2_ref, out_ref):
    d = _d_col(degp_ref[...], n, n_pad)
    o = z2_ref[...] - d * _psum(sp_ref[...], n, n_pad) + b2_ref[...]
    m = jnp.max(o, axis=1, keepdims=True)
    e = jnp.exp(o - m)
    out_ref[...] = (o - m) - jnp.log(jnp.sum(e, axis=1, keepdims=True))


def kernel(x, edge_index, W1, b1, W2, b2):
    n, in_c = x.shape
    e = edge_index.shape[1]
    hid = W1.shape[2]
    out_c = W2.shape[2]

    batch = 8
    c_chunks = batch * (-(-e // (_NW * _CHUNK * batch)))
    e_pad = _NW * c_chunks * _CHUNK
    n_pad = ((n + _CHUNK + 127) // 128) * 128

    rows = edge_index[0].astype(jnp.int32)
    cols = edge_index[1].astype(jnp.int32)
    pad = jnp.asarray(
        n + (np.arange(e_pad - e, dtype=np.int32) % _CHUNK), jnp.int32)
    shape4 = (_NW, c_chunks // batch, batch, _CHUNK)
    rows3 = jnp.concatenate([rows, pad]).reshape(shape4)
    cols3 = jnp.concatenate([cols, pad]).reshape(shape4)

    degp = _make_deg_kernel(n_pad, c_chunks, batch)(rows3)

    b1r = b1.reshape(1, hid)
    b2r = b2.reshape(1, out_c)

    z1, y1 = pl.pallas_call(
        _tc1a_body,
        out_shape=(jax.ShapeDtypeStruct((n, hid), jnp.float32),
                   jax.ShapeDtypeStruct((n, hid), jnp.float32)),
    )(x, W1[0], W1[1])

    g1p = pl.pallas_call(
        functools.partial(_tc1b_body, n, n_pad),
        out_shape=jax.ShapeDtypeStruct((n_pad, hid), jnp.float32),
    )(y1, degp)

    s1p = _make_spmm_kernel(n_pad, c_chunks, hid, batch)(g1p, rows3, cols3)

    z2, g2p = pl.pallas_call(
        functools.partial(_tc2_body, n, n_pad),
        out_shape=(jax.ShapeDtypeStruct((n, out_c), jnp.float32),
                   jax.ShapeDtypeStruct((n_pad, out_c), jnp.float32)),
    )(z1, s1p, degp, b1r, W2[0], W2[1])

    s2p = _make_spmm_kernel(n_pad, c_chunks, out_c, batch)(g2p, rows3,
                                                           cols3)

    return pl.pallas_call(
        functools.partial(_tc3_body, n, n_pad),
        out_shape=jax.ShapeDtypeStruct((n, out_c), jnp.float32),
    )(z2, s2p, degp, b2r)


# speedup vs baseline: 1.0342x; 1.0322x over previous
import functools

import jax
import jax.numpy as jnp
import numpy as np
from jax import lax
from jax.experimental import pallas as pl
from jax.experimental.pallas import tpu as pltpu
from jax.experimental.pallas import tpu_sc as plsc

_NW = 32
_NS = 16
_CHUNK = 128


def _make_deg_kernel(n_pad, c_chunks, batch):
    mesh = plsc.VectorSubcoreMesh(core_axis_name="c", subcore_axis_name="s")
    zrows = n_pad // _NS
    groups = c_chunks // batch
    brows = batch * _CHUNK

    @functools.partial(
        pl.kernel,
        mesh=mesh,
        out_type=jax.ShapeDtypeStruct((2 * n_pad, 16), jnp.float32),
        compiler_params=pltpu.CompilerParams(use_tc_tiling_on_sc=False),
        scratch_types=[
            pltpu.VMEM((groups, batch, _CHUNK), jnp.int32),
            pltpu.VMEM((_CHUNK, 16), jnp.float32),
            pltpu.VMEM((_CHUNK, 16), jnp.float32),
            pltpu.VMEM_SHARED((n_pad, 16), jnp.float32),
        ],
    )
    def k(rows_hbm, out_hbm, rows_v, ebuf, zbuf, acc):
        cid = lax.axis_index("c")
        sid = lax.axis_index("s")
        wid = cid * _NS + sid
        e0v = jnp.where(lax.iota(jnp.int32, 16) == 0, 1.0, 0.0)
        z16 = jnp.zeros((16,), jnp.float32)

        def fill(i, carry):
            ebuf[i, pl.ds(0, 16)] = e0v
            return carry

        lax.fori_loop(0, _CHUNK, fill, 0)

        def fillz(i, carry):
            zbuf[i, pl.ds(0, 16)] = z16
            return carry

        lax.fori_loop(0, _CHUNK, fillz, 0)

        def zstripe(k2, carry):
            pltpu.sync_copy(
                zbuf, acc.at[pl.ds(sid * zrows + k2 * _CHUNK, _CHUNK)])
            return carry

        lax.fori_loop(0, zrows // _CHUNK, zstripe, 0)
        pltpu.sync_copy(rows_hbm.at[wid], rows_v)
        plsc.subcore_barrier()

        def body(j, carry):
            for b in range(batch):
                pltpu.sync_copy(ebuf, acc.at[rows_v.at[j, b]], add=True)
            return carry

        lax.fori_loop(0, groups, body, 0)
        plsc.subcore_barrier()
        pltpu.sync_copy(acc.at[pl.ds(sid * zrows, zrows)],
                        out_hbm.at[pl.ds(cid * n_pad + sid * zrows, zrows)])

    return k


def _make_spmm_kernel(n_pad, c_chunks, feat, batch):
    mesh = plsc.VectorSubcoreMesh(core_axis_name="c", subcore_axis_name="s")
    zrows = n_pad // _NS
    groups = c_chunks // batch
    brows = batch * _CHUNK

    @functools.partial(
        pl.kernel,
        mesh=mesh,
        out_type=jax.ShapeDtypeStruct((2 * n_pad, feat), jnp.float32),
        compiler_params=pltpu.CompilerParams(use_tc_tiling_on_sc=False),
        scratch_types=[
            pltpu.VMEM((groups + 1, batch, _CHUNK), jnp.int32),
            pltpu.VMEM((groups, batch, _CHUNK), jnp.int32),
            pltpu.VMEM((_CHUNK, feat), jnp.float32),
            pltpu.VMEM((_CHUNK, feat), jnp.float32),
            pltpu.VMEM_SHARED((n_pad, feat), jnp.float32),
            pltpu.VMEM_SHARED((n_pad, feat), jnp.float32),
            pltpu.SemaphoreType.DMA,
            pltpu.SemaphoreType.DMA,
            pltpu.SemaphoreType.DMA,
            pltpu.SemaphoreType.DMA,
        ],
    )
    def k(g_hbm, rows_hbm, cols_hbm, out_hbm,
          rows_v, cols_v, gbuf0, gbuf1, acc, gtab, sem0, sem1, sem2, sem3):
        cid = lax.axis_index("c")
        sid = lax.axis_index("s")
        wid = cid * _NS + sid
        z16 = jnp.zeros((16,), jnp.float32)

        def fill(i, carry):
            for kk in range(feat // 16):
                gbuf0[i, pl.ds(16 * kk, 16)] = z16
            return carry

        lax.fori_loop(0, _CHUNK, fill, 0)

        def zstripe(k2, carry):
            pltpu.sync_copy(
                gbuf0.at[pl.ds(0, _CHUNK)],
                acc.at[pl.ds(sid * zrows + k2 * _CHUNK, _CHUNK)])
            return carry

        lax.fori_loop(0, zrows // _CHUNK, zstripe, 0)
        pltpu.sync_copy(rows_hbm.at[wid], rows_v.at[pl.ds(0, groups)])
        pltpu.sync_copy(cols_hbm.at[wid], cols_v)
        lane = lax.iota(jnp.int32, 16)
        for b in range(batch):
            for kk in range(_CHUNK // 16):
                rows_v[groups, b, pl.ds(16 * kk, 16)] = (
                    (n_pad - _CHUNK) + 16 * kk + lane)
        pltpu.sync_copy(g_hbm.at[pl.ds(sid * zrows, zrows)],
                        gtab.at[pl.ds(sid * zrows, zrows)])
        plsc.subcore_barrier()

        bufs = (gbuf0, gbuf1)
        gsems = (sem0, sem1)
        ssems = (sem2, sem3)

        def gat(j, b, t):
            pltpu.async_copy(gtab.at[rows_v.at[j, b]], bufs[t], gsems[t])

        def gwait(j, b, t):
            pltpu.make_async_copy(gtab.at[rows_v.at[j, b]], bufs[t],
                                  gsems[t]).wait()

        def sca(j, b, t):
            pltpu.async_copy(bufs[t], acc.at[cols_v.at[j, b]], ssems[t],
                             add=True)

        def swait(j, b, t):
            pltpu.make_async_copy(bufs[t], acc.at[cols_v.at[j, b]],
                                  ssems[t]).wait()

        def step(j, b, jn, bn):
            cur = b % 2
            nxt = 1 - cur
            gwait(j, b, cur)
            sca(j, b, cur)
            swait(j, b, nxt)
            gat(jn, bn, nxt)

        gat(0, 0, 0)
        gwait(0, 0, 0)
        sca(0, 0, 0)
        gat(0, 1, 1)
        for b in range(1, batch):
            jn, bn = (0, b + 1) if b + 1 < batch else (1, 0)
            if b == 1:
                gwait(0, 1, 1)
                sca(0, 1, 1)
                swait(0, 1, 0)
                gat(jn, bn, 0)
            else:
                step(0, b, jn, bn)

        def body(j, carry):
            for b in range(batch):
                jn, bn = (j, b + 1) if b + 1 < batch else (j + 1, 0)
                step(j, b, jn, bn)
            return carry

        lax.fori_loop(1, groups, body, 0)
        gwait(groups, 0, 0)
        swait(0, 0, (batch - 1) % 2)
        plsc.subcore_barrier()
        pltpu.sync_copy(acc.at[pl.ds(sid * zrows, zrows)],
                        out_hbm.at[pl.ds(cid * n_pad + sid * zrows, zrows)])

    return k


def _d_col(degp, n, n_pad):
    deg = degp[:n, 0:1] + degp[n_pad:n_pad + n, 0:1]
    return jnp.where(deg > 0.0, lax.rsqrt(deg), 0.0)


def _psum(sp, n, n_pad):
    return sp[:n] + sp[n_pad:n_pad + n]


def _tc1a_body(x_ref, w10_ref, w11_ref, z1_ref, y1_ref):
    x = x_ref[...]
    z1_ref[...] = jnp.dot(x, w10_ref[...], preferred_element_type=jnp.float32)
    y1_ref[...] = jnp.dot(x, w11_ref[...], preferred_element_type=jnp.float32)


def _tc1b_body(n, n_pad, y1_ref, degp_ref, g1_ref):
    d = _d_col(degp_ref[...], n, n_pad)
    g1_ref[0:n, :] = d * y1_ref[...]
    g1_ref[n:n_pad, :] = jnp.zeros((n_pad - n, y1_ref.shape[1]), jnp.float32)


def _tc2_body(n, n_pad, z1_ref, sp_ref, degp_ref, b1_ref, w20_ref, w21_ref,
              z2_ref, g2_ref):
    d = _d_col(degp_ref[...], n, n_pad)
    s1 = d * _psum(sp_ref[...], n, n_pad)
    h = jnp.maximum(z1_ref[...] - s1 + b1_ref[...], 0.0)
    z2_ref[...] = jnp.dot(h, w20_ref[...], preferred_element_type=jnp.float32)
    g2_ref[0:n, :] = d * jnp.dot(h, w21_ref[...],
                                 preferred_element_type=jnp.float32)
    g2_ref[n:n_pad, :] = jnp.zeros((n_pad - n, w21_ref.shape[1]), jnp.float32)


def _tc3_body(n, n_pad, z2_ref, sp_ref, degp_ref, b2_ref, out_ref):
    d = _d_col(degp_ref[...], n, n_pad)
    o = z2_ref[...] - d * _psum(sp_ref[...], n, n_pad) + b2_ref[...]
    m = jnp.max(o, axis=1, keepdims=True)
    e = jnp.exp(o - m)
    out_ref[...] = (o - m) - jnp.log(jnp.sum(e, axis=1, keepdims=True))


def kernel(x, edge_index, W1, b1, W2, b2):
    n, in_c = x.shape
    e = edge_index.shape[1]
    hid = W1.shape[2]
    out_c = W2.shape[2]

    batch = 8
    c_chunks = batch * (-(-e // (_NW * _CHUNK * batch)))
    e_pad = _NW * c_chunks * _CHUNK
    n_pad = ((n + _CHUNK + 127) // 128) * 128

    rows = edge_index[0].astype(jnp.int32)
    cols = edge_index[1].astype(jnp.int32)
    pad = jnp.asarray(
        n + (np.arange(e_pad - e, dtype=np.int32) % _CHUNK), jnp.int32)
    shape4 = (_NW, c_chunks // batch, batch, _CHUNK)
    rows3 = jnp.concatenate([rows, pad]).reshape(shape4)
    cols3 = jnp.concatenate([cols, pad]).reshape(shape4)

    degp = _make_deg_kernel(n_pad, c_chunks, batch)(rows3)

    b1r = b1.reshape(1, hid)
    b2r = b2.reshape(1, out_c)

    z1, y1 = pl.pallas_call(
        _tc1a_body,
        out_shape=(jax.ShapeDtypeStruct((n, hid), jnp.float32),
                   jax.ShapeDtypeStruct((n, hid), jnp.float32)),
    )(x, W1[0], W1[1])

    g1p = pl.pallas_call(
        functools.partial(_tc1b_body, n, n_pad),
        out_shape=jax.ShapeDtypeStruct((n_pad, hid), jnp.float32),
    )(y1, degp)

    s1p = _make_spmm_kernel(n_pad, c_chunks, hid, batch)(g1p, rows3, cols3)

    z2, g2p = pl.pallas_call(
        functools.partial(_tc2_body, n, n_pad),
        out_shape=(jax.ShapeDtypeStruct((n, out_c), jnp.float32),
                   jax.ShapeDtypeStruct((n_pad, out_c), jnp.float32)),
    )(z1, s1p, degp, b1r, W2[0], W2[1])

    s2p = _make_spmm_kernel(n_pad, c_chunks, out_c, batch)(g2p, rows3,
                                                           cols3)

    return pl.pallas_call(
        functools.partial(_tc3_body, n, n_pad),
        out_shape=jax.ShapeDtypeStruct((n, out_c), jnp.float32),
    )(z2, s2p, degp, b2r)
